# baseline (speedup 1.0000x reference)
import functools

import jax
import jax.numpy as jnp
from jax import lax
from jax.experimental import pallas as pl
from jax.experimental.pallas import tpu as pltpu
from jax.experimental.pallas import tpu_sc as plsc

_NC = 2
_NS = 16
_NW = _NC * _NS
_CHUNK = 64
_SLOTS = 2


@functools.partial(jax.jit, static_argnums=(2, 3))
def _sc_embedding_gather(tokens_3d, table, b, d):
    b_per_w = b // _NW
    n_chunks = b_per_w // _CHUNK
    mesh = plsc.VectorSubcoreMesh(core_axis_name="c", subcore_axis_name="s")

    @functools.partial(
        pl.kernel,
        mesh=mesh,
        out_type=jax.ShapeDtypeStruct((b, d), jnp.float32),
        scratch_types=(
            [pltpu.VMEM((n_chunks, _CHUNK), jnp.int32)]
            + [pltpu.VMEM((_CHUNK, d), jnp.float32) for _ in range(_SLOTS)]
            + [pltpu.VMEM_SHARED((_NS, _SLOTS, _CHUNK, d), jnp.float32)]
            + [pltpu.SemaphoreType.DMA for _ in range(2 * _SLOTS)]
        ),
    )
    def k(tok_hbm, tab_hbm, out_hbm, idx_v, *rest):
        rows = rest[:_SLOTS]
        spbuf = rest[_SLOTS]
        gsem = rest[_SLOTS + 1:2 * _SLOTS + 1]
        wsem = rest[2 * _SLOTS + 1:]
        sid = lax.axis_index("s")
        wid = sid * _NC + lax.axis_index("c")
        base = wid * b_per_w

        pltpu.sync_copy(tok_hbm.at[wid], idx_v)

        def gather_start(c, p):
            pltpu.make_async_copy(tab_hbm.at[idx_v.at[c]], rows[p], gsem[p]).start()

        def gather_wait(p):
            pltpu.make_async_copy(tab_hbm.at[idx_v.at[0]], rows[p], gsem[p]).wait()

        def write_start(c, p):
            pltpu.sync_copy(rows[p], spbuf.at[sid, p])
            pltpu.make_async_copy(
                spbuf.at[sid, p], out_hbm.at[pl.ds(base + c * _CHUNK, _CHUNK)],
                wsem[p],
            ).start()

        def write_wait(p):
            pltpu.make_async_copy(
                spbuf.at[sid, 0], out_hbm.at[pl.ds(base, _CHUNK)], wsem[p]
            ).wait()

        for p in range(_SLOTS):
            gather_start(p, p)

        def body(j, carry):
            c0 = _SLOTS * j
            for p in range(_SLOTS):
                gather_wait(p)
                write_start(c0 + p, p)
            for p in range(_SLOTS):
                write_wait(p)
                gather_start(lax.min(c0 + _SLOTS + p, n_chunks - 1), p)
            return carry

        lax.fori_loop(0, n_chunks // _SLOTS, body, 0)
        for p in range(_SLOTS):
            gather_wait(p)

    return k(tokens_3d, table)


def kernel(integer_tokens, token_embedding):
    bsz, seq = integer_tokens.shape
    d = token_embedding.shape[1]
    n = bsz * seq
    tok3d = integer_tokens.reshape(_NW, n // (_NW * _CHUNK), _CHUNK)
    out = _sc_embedding_gather(tok3d, token_embedding, n, d)
    return out.reshape(bsz, seq, d)


# speedup vs baseline: 1.0214x; 1.0214x over previous
import functools

import jax
import jax.numpy as jnp
from jax import lax
from jax.experimental import pallas as pl
from jax.experimental.pallas import tpu as pltpu
from jax.experimental.pallas import tpu_sc as plsc

_NC = 2
_NS = 16
_NW = _NC * _NS
_CHUNK = 128


@functools.partial(jax.jit, static_argnums=(2, 3))
def _sc_embedding_gather(tokens_3d, table, b, d):
    b_per_w = b // _NW
    n_chunks = b_per_w // _CHUNK
    mesh = plsc.VectorSubcoreMesh(core_axis_name="c", subcore_axis_name="s")

    @functools.partial(
        pl.kernel,
        mesh=mesh,
        out_type=jax.ShapeDtypeStruct((b, d), jnp.float32),
        scratch_types=[
            pltpu.VMEM((n_chunks, _CHUNK), jnp.int32),
            pltpu.VMEM((_CHUNK, d), jnp.float32),
            pltpu.VMEM((_CHUNK, d), jnp.float32),
            pltpu.SemaphoreType.DMA,
            pltpu.SemaphoreType.DMA,
        ],
    )
    def k(tok_hbm, tab_hbm, out_hbm, idx_v, rows_a, rows_b, sem_a, sem_b):
        wid = lax.axis_index("s") * _NC + lax.axis_index("c")
        base = wid * b_per_w

        pltpu.sync_copy(tok_hbm.at[wid], idx_v)

        def gather_start(c, buf, sem):
            pltpu.make_async_copy(tab_hbm.at[idx_v.at[c]], buf, sem).start()

        def gather_wait(buf, sem):
            pltpu.make_async_copy(tab_hbm.at[idx_v.at[0]], buf, sem).wait()

        def write_out(c, buf):
            pltpu.sync_copy(buf, out_hbm.at[pl.ds(base + c * _CHUNK, _CHUNK)])

        gather_start(0, rows_a, sem_a)

        def body(j, carry):
            c0 = 2 * j
            c1 = c0 + 1
            gather_wait(rows_a, sem_a)
            gather_start(c1, rows_b, sem_b)
            write_out(c0, rows_a)
            gather_wait(rows_b, sem_b)
            gather_start(lax.min(c0 + 2, n_chunks - 1), rows_a, sem_a)
            write_out(c1, rows_b)
            return carry

        lax.fori_loop(0, n_chunks // 2, body, 0)
        gather_wait(rows_a, sem_a)

    return k(tokens_3d, table)


def kernel(integer_tokens, token_embedding):
    bsz, seq = integer_tokens.shape
    d = token_embedding.shape[1]
    n = bsz * seq
    tok3d = integer_tokens.reshape(_NW, n // (_NW * _CHUNK), _CHUNK)
    out = _sc_embedding_gather(tok3d, token_embedding, n, d)
    return out.reshape(bsz, seq, d)
